# baseline (speedup 1.0000x reference)
import functools

import jax
import jax.numpy as jnp
import numpy as np
from jax.experimental import pallas as pl
from jax.experimental.pallas import tpu as pltpu

_EPS = 1e-3
_B, _N, _F, _T, _H = 2, 384, 128, 8, 64
_INV_DEG = 1.0 / (_N - 1)
_RSQ = 1.0 / np.sqrt(1.0 + _EPS)

_LAYER_DIMS = (_T, _H, _F, _H, 2 * _H, _H, 3 * _H, _H,
               2 * _H, _H, 3 * _H, _H, _H, _H)
_WSHAPES = []
for _d in _LAYER_DIMS:
    _WSHAPES += [(1, _d), (1, _d), (_d, _H), (1, _H)]
_WSHAPES += [(_H, _F), (1, _F)]
_NW = len(_WSHAPES)


def _gnn_body(time_ref, p_ref, *refs):
    w_hbm = refs[:_NW]
    out_ref = refs[_NW]
    w_vmem = refs[_NW + 1:2 * _NW + 1]
    sems = refs[2 * _NW + 1]

    def copy(i):
        return pltpu.make_async_copy(w_hbm[i], w_vmem[i], sems.at[i])

    for i in range(_NW):
        copy(i).start()

    (tm1, tm2, pre1, pre2, c1p1, c1p2, c1u1, c1u2,
     c2p1, c2p2, c2u1, c2u2, post1, post2) = (
        [w_vmem[4 * i:4 * i + 4] for i in range(14)])
    lg_w_ref, lg_b_ref = w_vmem[56], w_vmem[57]

    dot = functools.partial(jnp.dot, precision=jax.lax.Precision.DEFAULT,
                            preferred_element_type=jnp.float32)
    gelu = jax.nn.gelu

    def bdiag(w):
        d, u = w.shape
        z = jnp.zeros((d, u), jnp.float32)
        return jnp.concatenate(
            [jnp.concatenate([w, z], axis=1),
             jnp.concatenate([z, w], axis=1)], axis=0)

    def dup(v):
        return jnp.concatenate([v, v], axis=1)

    def pair(v):
        return jnp.concatenate([v[0:1], v[1:2]], axis=1)

    def bn_bl(x, lp, lo, hi):
        return (x * dup(lp[0][:, lo:hi] * _RSQ) + dup(lp[1][:, lo:hi]))

    def bn_t(v, lp, lo, hi):
        return v * (lp[0][:, lo:hi] * _RSQ) + lp[1][:, lo:hi]

    def layer_bl(x, lp):
        d = lp[2].shape[0]
        return gelu(dot(bn_bl(x, lp, 0, d), bdiag(lp[2][...]))
                    + dup(lp[3][...]))

    for i in range(16):
        copy(i).wait()

    t = gelu(dot(bn_t(time_ref[...], tm1, 0, _T), tm1[2][...]) + tm1[3][...])
    t = gelu(dot(bn_t(t, tm2, 0, _H), tm2[2][...]) + tm2[3][...])

    x = jnp.concatenate([p_ref[0], p_ref[1]], axis=1)
    x = layer_bl(x, pre1)
    x = layer_bl(x, pre2)

    for ci, (pl1, pl2, ul1, ul2) in enumerate(
            ((c1p1, c1p2, c1u1, c1u2), (c2p1, c2p2, c2u1, c2u2))):
        for i in range(16 + 16 * ci, 32 + 16 * ci):
            copy(i).wait()
        tb = dot(bn_t(t, pl1, _H, 2 * _H), pl1[2][_H:, :]) + pl1[3][...]
        g = gelu(dot(bn_bl(x, pl1, 0, _H), bdiag(pl1[2][:_H, :])) + pair(tb))
        g = layer_bl(g, pl2)
        s = jnp.sum(g, axis=0, keepdims=True)
        agg = (s - g) * _INV_DEG
        utb = (dot(bn_t(t, ul1, 2 * _H, 3 * _H), ul1[2][2 * _H:, :])
               + ul1[3][...])
        u = gelu(dot(bn_bl(x, ul1, 0, _H), bdiag(ul1[2][:_H, :]))
                 + dot(bn_bl(agg, ul1, _H, 2 * _H),
                       bdiag(ul1[2][_H:2 * _H, :]))
                 + pair(utb))
        u = layer_bl(u, ul2)
        x = x + u

    for i in range(48, _NW):
        copy(i).wait()

    x = layer_bl(x, post1)
    x = layer_bl(x, post2)
    o = dot(x, bdiag(lg_w_ref[...])) + dup(lg_b_ref[...])
    out_ref[0] = o[:, 0:_F]
    out_ref[1] = o[:, _F:2 * _F]


def kernel(p, time, params):
    weights = []
    for key in ("time_mlp", "pre", "c1_prep", "c1_upd", "c2_prep", "c2_upd",
                "post"):
        for lay in params[key]:
            weights += [lay["gamma"][None, :], lay["beta"][None, :],
                        lay["W"], lay["b"][None, :]]
    weights.append(params["logits_W"])
    weights.append(params["logits_b"][None, :])

    vmem = pl.BlockSpec(memory_space=pltpu.MemorySpace.VMEM)
    hbm = pl.BlockSpec(memory_space=pltpu.MemorySpace.HBM)
    out = pl.pallas_call(
        _gnn_body,
        in_specs=[vmem, vmem] + [hbm] * _NW,
        out_specs=vmem,
        out_shape=jax.ShapeDtypeStruct((_B, _N, _F), jnp.float32),
        scratch_shapes=([pltpu.VMEM(s, jnp.float32) for s in _WSHAPES]
                       + [pltpu.SemaphoreType.DMA((_NW,))]),
    )(time, p, *weights)
    return out
